# initial kernel scaffold (unmeasured)
import jax
import jax.numpy as jnp
from jax import lax
from jax.experimental import pallas as pl
from jax.experimental.pallas import tpu as pltpu

N_DEV = 32
E_LOC = 4
N_EXP = 128
CAP = 48
CHUNK = E_LOC * CAP


def kernel(x, router_W, route_idx, expert_W, shared_W):
    n_tok, d = x.shape
    h = shared_W.shape[1]

    scores = x @ router_W
    probs = jax.nn.softmax(scores, axis=-1)
    p_tok = jnp.take_along_axis(probs, route_idx, axis=1)
    e = route_idx[:, 0]
    onehot = (e[:, None] == jnp.arange(N_EXP)[None, :]).astype(jnp.int32)
    pos = jnp.take_along_axis(
        jnp.cumsum(onehot, axis=0) - onehot, route_idx, axis=1
    )[:, 0]
    k = jnp.where(pos < CAP, e * CAP + pos, -1).astype(jnp.int32)

    k_row = k[None, :]
    k_col = k[:, None]
    x_bf = x.astype(jnp.bfloat16)
    p_bf = p_tok.astype(jnp.bfloat16)
    sW = shared_W.astype(jnp.bfloat16)
    eW = expert_W.astype(jnp.bfloat16)

    def body(x_ref, p_ref, krow_ref, kcol_ref, sW_ref, eW_ref, out_ref,
             disp_ref, r_ref, y_ref, z_ref,
             send1, recv1, send2, recv2, cp_sem1, cp_sem2):
        me = lax.axis_index("i")
        xs = x_ref[...] * p_ref[...]
        krow = krow_ref[...]

        def build(c, carry):
            oh = (krow == lax.broadcasted_iota(jnp.int32, (CHUNK, n_tok), 0)
                  + c * CHUNK).astype(jnp.bfloat16)
            chunk = jnp.dot(oh, xs, preferred_element_type=jnp.float32)
            disp_ref[pl.ds(c * E_LOC, E_LOC)] = (
                chunk.astype(jnp.bfloat16).reshape(E_LOC, CAP, d))
            return carry
        lax.fori_loop(0, N_DEV, build, 0)

        cp1 = pltpu.make_async_copy(
            disp_ref.at[pl.ds(me * E_LOC, E_LOC)], r_ref.at[me], cp_sem1)
        cp1.start()

        def send_disp(t, carry):
            dd = lax.rem(me + 1 + t, N_DEV)
            pltpu.make_async_remote_copy(
                src_ref=disp_ref.at[pl.ds(dd * E_LOC, E_LOC)],
                dst_ref=r_ref.at[me],
                send_sem=send1.at[dd],
                recv_sem=recv1.at[me],
                device_id=dd,
                device_id_type=pl.DeviceIdType.LOGICAL,
            ).start()
            return carry
        lax.fori_loop(0, N_DEV - 1, send_disp, 0)

        sh = jnp.dot(x_ref[...], sW_ref[...],
                     preferred_element_type=jnp.float32)
        out_ref[...] = sh.astype(jnp.bfloat16)

        def wait_disp(t, carry):
            s = lax.rem(me + 1 + t, N_DEV)
            pltpu.make_async_remote_copy(
                src_ref=disp_ref.at[pl.ds(0, E_LOC)],
                dst_ref=r_ref.at[s],
                send_sem=send1.at[s],
                recv_sem=recv1.at[s],
                device_id=me,
                device_id_type=pl.DeviceIdType.LOGICAL,
            ).wait_recv()
            return carry
        lax.fori_loop(0, N_DEV - 1, wait_disp, 0)
        cp1.wait()

        for le in range(E_LOC):
            a = r_ref[:, le, :, :].reshape(N_DEV * CAP, d)
            yv = jnp.dot(a, eW_ref[le], preferred_element_type=jnp.float32)
            y_ref[:, le, :, :] = yv.astype(jnp.bfloat16).reshape(N_DEV, CAP, h)

        cp2 = pltpu.make_async_copy(
            y_ref.at[me], z_ref.at[pl.ds(me * E_LOC, E_LOC)], cp_sem2)
        cp2.start()

        def send_ret(t, carry):
            dd = lax.rem(me + 1 + t, N_DEV)
            pltpu.make_async_remote_copy(
                src_ref=y_ref.at[dd],
                dst_ref=z_ref.at[pl.ds(me * E_LOC, E_LOC)],
                send_sem=send2.at[dd],
                recv_sem=recv2.at[me],
                device_id=dd,
                device_id_type=pl.DeviceIdType.LOGICAL,
            ).start()
            return carry
        lax.fori_loop(0, N_DEV - 1, send_ret, 0)

        def wait_ret(t, carry):
            s = lax.rem(me + 1 + t, N_DEV)
            pltpu.make_async_remote_copy(
                src_ref=y_ref.at[0],
                dst_ref=z_ref.at[pl.ds(s * E_LOC, E_LOC)],
                send_sem=send2.at[s],
                recv_sem=recv2.at[s],
                device_id=me,
                device_id_type=pl.DeviceIdType.LOGICAL,
            ).wait_recv()
            return carry
        lax.fori_loop(0, N_DEV - 1, wait_ret, 0)
        cp2.wait()

        kcol = kcol_ref[...]
        def combine(c, acc):
            oh = (kcol == lax.broadcasted_iota(jnp.int32, (n_tok, CHUNK), 1)
                  + c * CHUNK).astype(jnp.bfloat16)
            zc = z_ref[pl.ds(c * E_LOC, E_LOC)].reshape(CHUNK, h)
            return acc + jnp.dot(
                oh, zc, preferred_element_type=jnp.float32).astype(jnp.bfloat16)
        acc = lax.fori_loop(
            0, N_DEV, combine, jnp.zeros((n_tok, h), jnp.bfloat16))
        out_ref[...] = out_ref[...] + acc

        def wait_sends(t, carry):
            dd = lax.rem(me + 1 + t, N_DEV)
            pltpu.make_async_remote_copy(
                src_ref=disp_ref.at[pl.ds(0, E_LOC)],
                dst_ref=r_ref.at[0],
                send_sem=send1.at[dd],
                recv_sem=recv1.at[0],
                device_id=me,
                device_id_type=pl.DeviceIdType.LOGICAL,
            ).wait_send()
            pltpu.make_async_remote_copy(
                src_ref=y_ref.at[0],
                dst_ref=z_ref.at[pl.ds(0, E_LOC)],
                send_sem=send2.at[dd],
                recv_sem=recv2.at[0],
                device_id=me,
                device_id_type=pl.DeviceIdType.LOGICAL,
            ).wait_send()
            return carry
        lax.fori_loop(0, N_DEV - 1, wait_sends, 0)

    out_bf = pl.pallas_call(
        body,
        out_shape=jax.ShapeDtypeStruct((n_tok, h), jnp.bfloat16),
        in_specs=[pl.BlockSpec(memory_space=pltpu.VMEM)] * 6,
        out_specs=pl.BlockSpec(memory_space=pltpu.VMEM),
        scratch_shapes=[
            pltpu.VMEM((N_EXP, CAP, d), jnp.bfloat16),
            pltpu.VMEM((N_DEV, E_LOC, CAP, d), jnp.bfloat16),
            pltpu.VMEM((N_DEV, E_LOC, CAP, h), jnp.bfloat16),
            pltpu.VMEM((N_EXP, CAP, h), jnp.bfloat16),
            pltpu.SemaphoreType.DMA((N_DEV,)),
            pltpu.SemaphoreType.DMA((N_DEV,)),
            pltpu.SemaphoreType.DMA((N_DEV,)),
            pltpu.SemaphoreType.DMA((N_DEV,)),
            pltpu.SemaphoreType.DMA,
            pltpu.SemaphoreType.DMA,
        ],
        compiler_params=pltpu.CompilerParams(has_side_effects=True),
    )(x_bf, p_bf, k_row, k_col, sW, eW)

    return out_bf.astype(jnp.float32)


# baseline (device time: 366953 ns/iter reference)
import jax
import jax.numpy as jnp
from jax import lax
from jax.experimental import pallas as pl
from jax.experimental.pallas import tpu as pltpu

N_DEV = 32
E_LOC = 4
N_EXP = 128
CAP = 48
CHUNK = E_LOC * CAP


def kernel(x, router_W, route_idx, expert_W, shared_W):
    n_tok, d = x.shape
    h = shared_W.shape[1]

    scores = x @ router_W
    probs = jax.nn.softmax(scores, axis=-1)
    p_tok = jnp.take_along_axis(probs, route_idx, axis=1)
    e = route_idx[:, 0]
    onehot = (e[:, None] == jnp.arange(N_EXP)[None, :]).astype(jnp.int32)
    pos = jnp.take_along_axis(
        jnp.cumsum(onehot, axis=0) - onehot, route_idx, axis=1
    )[:, 0]
    k = jnp.where(pos < CAP, e * CAP + pos, -1).astype(jnp.int32)

    k_row = k[None, :]
    k_col = k[:, None]
    x_bf = x.astype(jnp.bfloat16)
    p_bf = p_tok.astype(jnp.bfloat16)
    sW = shared_W.astype(jnp.bfloat16)
    eW = expert_W.astype(jnp.bfloat16)

    def body(x_ref, p_ref, krow_ref, kcol_ref, sW_ref, eW_ref, out_ref,
             disp_ref, r_ref, y_ref, z_ref,
             send1, recv1, send2, recv2, cp_sem1, cp_sem2):
        me = lax.axis_index("i")
        xs = x_ref[...] * p_ref[...]
        krow = krow_ref[...]

        def build(c, carry):
            oh = (krow == lax.broadcasted_iota(jnp.int32, (CHUNK, n_tok), 0)
                  + c * CHUNK).astype(jnp.bfloat16)
            chunk = jnp.dot(oh, xs, preferred_element_type=jnp.float32)
            disp_ref[pl.ds(c * E_LOC, E_LOC)] = (
                chunk.astype(jnp.bfloat16).reshape(E_LOC, CAP, d))
            return carry
        lax.fori_loop(0, N_DEV, build, 0)

        cp1 = pltpu.make_async_copy(
            disp_ref.at[pl.ds(me * E_LOC, E_LOC)], r_ref.at[me], cp_sem1)
        cp1.start()

        def send_disp(t, carry):
            dd = lax.rem(me + 1 + t, N_DEV)
            pltpu.make_async_remote_copy(
                src_ref=disp_ref.at[pl.ds(dd * E_LOC, E_LOC)],
                dst_ref=r_ref.at[me],
                send_sem=send1.at[dd],
                recv_sem=recv1.at[me],
                device_id=dd,
                device_id_type=pl.DeviceIdType.LOGICAL,
            ).start()
            return carry
        lax.fori_loop(0, N_DEV - 1, send_disp, 0)

        sh = jnp.dot(x_ref[...], sW_ref[...],
                     preferred_element_type=jnp.float32)
        out_ref[...] = sh.astype(jnp.bfloat16)

        def wait_disp(t, carry):
            s = lax.rem(me + 1 + t, N_DEV)
            pltpu.make_async_remote_copy(
                src_ref=disp_ref.at[pl.ds(0, E_LOC)],
                dst_ref=r_ref.at[s],
                send_sem=send1.at[s],
                recv_sem=recv1.at[s],
                device_id=me,
                device_id_type=pl.DeviceIdType.LOGICAL,
            ).wait_recv()
            return carry
        lax.fori_loop(0, N_DEV - 1, wait_disp, 0)
        cp1.wait()

        for le in range(E_LOC):
            a = r_ref[:, le, :, :].reshape(N_DEV * CAP, d)
            yv = jnp.dot(a, eW_ref[le], preferred_element_type=jnp.float32)
            y_ref[:, le, :, :] = yv.astype(jnp.bfloat16).reshape(N_DEV, CAP, h)

        cp2 = pltpu.make_async_copy(
            y_ref.at[me], z_ref.at[pl.ds(me * E_LOC, E_LOC)], cp_sem2)
        cp2.start()

        def send_ret(t, carry):
            dd = lax.rem(me + 1 + t, N_DEV)
            pltpu.make_async_remote_copy(
                src_ref=y_ref.at[dd],
                dst_ref=z_ref.at[pl.ds(me * E_LOC, E_LOC)],
                send_sem=send2.at[dd],
                recv_sem=recv2.at[me],
                device_id=dd,
                device_id_type=pl.DeviceIdType.LOGICAL,
            ).start()
            return carry
        lax.fori_loop(0, N_DEV - 1, send_ret, 0)

        def wait_ret(t, carry):
            s = lax.rem(me + 1 + t, N_DEV)
            pltpu.make_async_remote_copy(
                src_ref=y_ref.at[0],
                dst_ref=z_ref.at[pl.ds(s * E_LOC, E_LOC)],
                send_sem=send2.at[s],
                recv_sem=recv2.at[s],
                device_id=me,
                device_id_type=pl.DeviceIdType.LOGICAL,
            ).wait_recv()
            return carry
        lax.fori_loop(0, N_DEV - 1, wait_ret, 0)
        cp2.wait()

        kcol = kcol_ref[...]
        def combine(c, carry):
            oh = (kcol == lax.broadcasted_iota(jnp.int32, (n_tok, CHUNK), 1)
                  + c * CHUNK).astype(jnp.bfloat16)
            zc = z_ref[pl.ds(c * E_LOC, E_LOC)].reshape(CHUNK, h)
            out_ref[...] = out_ref[...] + jnp.dot(
                oh, zc, preferred_element_type=jnp.float32).astype(jnp.bfloat16)
            return carry
        lax.fori_loop(0, N_DEV, combine, 0)

        def wait_sends(t, carry):
            dd = lax.rem(me + 1 + t, N_DEV)
            pltpu.make_async_remote_copy(
                src_ref=disp_ref.at[pl.ds(0, E_LOC)],
                dst_ref=r_ref.at[0],
                send_sem=send1.at[dd],
                recv_sem=recv1.at[0],
                device_id=me,
                device_id_type=pl.DeviceIdType.LOGICAL,
            ).wait_send()
            pltpu.make_async_remote_copy(
                src_ref=y_ref.at[0],
                dst_ref=z_ref.at[pl.ds(0, E_LOC)],
                send_sem=send2.at[dd],
                recv_sem=recv2.at[0],
                device_id=me,
                device_id_type=pl.DeviceIdType.LOGICAL,
            ).wait_send()
            return carry
        lax.fori_loop(0, N_DEV - 1, wait_sends, 0)

    out_bf = pl.pallas_call(
        body,
        out_shape=jax.ShapeDtypeStruct((n_tok, h), jnp.bfloat16),
        in_specs=[pl.BlockSpec(memory_space=pltpu.VMEM)] * 6,
        out_specs=pl.BlockSpec(memory_space=pltpu.VMEM),
        scratch_shapes=[
            pltpu.VMEM((N_EXP, CAP, d), jnp.bfloat16),
            pltpu.VMEM((N_DEV, E_LOC, CAP, d), jnp.bfloat16),
            pltpu.VMEM((N_DEV, E_LOC, CAP, h), jnp.bfloat16),
            pltpu.VMEM((N_EXP, CAP, h), jnp.bfloat16),
            pltpu.SemaphoreType.DMA((N_DEV,)),
            pltpu.SemaphoreType.DMA((N_DEV,)),
            pltpu.SemaphoreType.DMA((N_DEV,)),
            pltpu.SemaphoreType.DMA((N_DEV,)),
            pltpu.SemaphoreType.DMA,
            pltpu.SemaphoreType.DMA,
        ],
        compiler_params=pltpu.CompilerParams(
            has_side_effects=True,
            vmem_limit_bytes=52 * 1024 * 1024,
        ),
    )(x_bf, p_bf, k_row, k_col, sW, eW)

    return out_bf.astype(jnp.float32)


# device time: 322287 ns/iter; 1.1386x vs baseline; 1.1386x over previous
import jax
import jax.numpy as jnp
from jax import lax
from jax.experimental import pallas as pl
from jax.experimental.pallas import tpu as pltpu

N_DEV = 32
E_LOC = 4
N_EXP = 128
CAP = 48
CHUNK = E_LOC * CAP
BLK = 8
N_BLK = N_DEV // BLK


def kernel(x, router_W, route_idx, expert_W, shared_W):
    n_tok, d = x.shape
    h = shared_W.shape[1]

    scores = x @ router_W
    probs = jax.nn.softmax(scores, axis=-1)
    e = route_idx[:, 0]
    onehot = (e[:, None] == jnp.arange(N_EXP)[None, :])
    p_tok = jnp.sum(probs * onehot, axis=1, keepdims=True)
    oh_i = onehot.astype(jnp.int32)
    pos = jnp.sum((jnp.cumsum(oh_i, axis=0) - oh_i) * oh_i, axis=1)
    k = jnp.where(pos < CAP, e * CAP + pos, -1).astype(jnp.int32)

    k_row = k[None, :]
    k_col = k[:, None]
    x_bf = x.astype(jnp.bfloat16)
    p_bf = p_tok.astype(jnp.bfloat16)
    sW = shared_W.astype(jnp.bfloat16)
    eW = expert_W.astype(jnp.bfloat16)

    def body(x_ref, p_ref, krow_ref, kcol_ref, sW_ref, eW_ref, out_ref,
             disp_ref, r_ref, y_ref, z_ref,
             send1, recv1, send2, recv2, cp_sem1, cp_sem2):
        me = lax.axis_index("i")
        xs = x_ref[...] * p_ref[...]
        krow = krow_ref[...]

        def build_send(t, carry):
            dd = lax.rem(me + 1 + t, N_DEV)
            oh = (krow == lax.broadcasted_iota(jnp.int32, (CHUNK, n_tok), 0)
                  + dd * CHUNK).astype(jnp.bfloat16)
            chunk = jnp.dot(oh, xs, preferred_element_type=jnp.float32)
            disp_ref[pl.ds(dd * E_LOC, E_LOC)] = (
                chunk.astype(jnp.bfloat16).reshape(E_LOC, CAP, d))

            @pl.when(dd != me)
            def _():
                pltpu.make_async_remote_copy(
                    src_ref=disp_ref.at[pl.ds(dd * E_LOC, E_LOC)],
                    dst_ref=r_ref.at[me],
                    send_sem=send1.at[dd],
                    recv_sem=recv1.at[me],
                    device_id=dd,
                    device_id_type=pl.DeviceIdType.LOGICAL,
                ).start()

            @pl.when(dd == me)
            def _():
                cp = pltpu.make_async_copy(
                    disp_ref.at[pl.ds(dd * E_LOC, E_LOC)],
                    r_ref.at[dd], cp_sem1)
                cp.start()
                cp.wait()
            return carry
        lax.fori_loop(0, N_DEV, build_send, 0)

        sh = jnp.dot(x_ref[...], sW_ref[...],
                     preferred_element_type=jnp.float32)
        out_ref[...] = sh.astype(jnp.bfloat16)

        for b in range(N_BLK):
            def wait_disp(j, carry):
                s = b * BLK + j

                @pl.when(s != me)
                def _():
                    pltpu.make_async_remote_copy(
                        src_ref=disp_ref.at[pl.ds(0, E_LOC)],
                        dst_ref=r_ref.at[s],
                        send_sem=send1.at[s],
                        recv_sem=recv1.at[s],
                        device_id=me,
                        device_id_type=pl.DeviceIdType.LOGICAL,
                    ).wait_recv()
                return carry
            lax.fori_loop(0, BLK, wait_disp, 0)

            for le in range(E_LOC):
                a = r_ref[pl.ds(b * BLK, BLK), le, :, :].reshape(BLK * CAP, d)
                yv = jnp.dot(a, eW_ref[le],
                             preferred_element_type=jnp.float32)
                y_ref[pl.ds(b * BLK, BLK), le, :, :] = (
                    yv.astype(jnp.bfloat16).reshape(BLK, CAP, h))

            def send_ret(j, carry):
                s = b * BLK + j

                @pl.when(s != me)
                def _():
                    pltpu.make_async_remote_copy(
                        src_ref=y_ref.at[s],
                        dst_ref=z_ref.at[pl.ds(me * E_LOC, E_LOC)],
                        send_sem=send2.at[s],
                        recv_sem=recv2.at[me],
                        device_id=s,
                        device_id_type=pl.DeviceIdType.LOGICAL,
                    ).start()

                @pl.when(s == me)
                def _():
                    cp = pltpu.make_async_copy(
                        y_ref.at[s], z_ref.at[pl.ds(s * E_LOC, E_LOC)],
                        cp_sem2)
                    cp.start()
                    cp.wait()
                return carry
            lax.fori_loop(0, BLK, send_ret, 0)

        kcol = kcol_ref[...]
        for b in range(N_BLK):
            def wait_ret(j, carry):
                s = b * BLK + j

                @pl.when(s != me)
                def _():
                    pltpu.make_async_remote_copy(
                        src_ref=y_ref.at[0],
                        dst_ref=z_ref.at[pl.ds(s * E_LOC, E_LOC)],
                        send_sem=send2.at[s],
                        recv_sem=recv2.at[s],
                        device_id=me,
                        device_id_type=pl.DeviceIdType.LOGICAL,
                    ).wait_recv()
                return carry
            lax.fori_loop(0, BLK, wait_ret, 0)

            def combine(c, carry):
                cc = b * BLK + c
                oh = (kcol == lax.broadcasted_iota(
                    jnp.int32, (n_tok, CHUNK), 1)
                    + cc * CHUNK).astype(jnp.bfloat16)
                zc = z_ref[pl.ds(cc * E_LOC, E_LOC)].reshape(CHUNK, h)
                out_ref[...] = out_ref[...] + jnp.dot(
                    oh, zc,
                    preferred_element_type=jnp.float32).astype(jnp.bfloat16)
                return carry
            lax.fori_loop(0, BLK, combine, 0)

        def wait_sends(t, carry):
            dd = lax.rem(me + 1 + t, N_DEV)
            pltpu.make_async_remote_copy(
                src_ref=disp_ref.at[pl.ds(0, E_LOC)],
                dst_ref=r_ref.at[0],
                send_sem=send1.at[dd],
                recv_sem=recv1.at[0],
                device_id=me,
                device_id_type=pl.DeviceIdType.LOGICAL,
            ).wait_send()
            pltpu.make_async_remote_copy(
                src_ref=y_ref.at[0],
                dst_ref=z_ref.at[pl.ds(0, E_LOC)],
                send_sem=send2.at[dd],
                recv_sem=recv2.at[0],
                device_id=me,
                device_id_type=pl.DeviceIdType.LOGICAL,
            ).wait_send()
            return carry
        lax.fori_loop(0, N_DEV - 1, wait_sends, 0)

    out_bf = pl.pallas_call(
        body,
        out_shape=jax.ShapeDtypeStruct((n_tok, h), jnp.bfloat16),
        in_specs=[pl.BlockSpec(memory_space=pltpu.VMEM)] * 6,
        out_specs=pl.BlockSpec(memory_space=pltpu.VMEM),
        scratch_shapes=[
            pltpu.VMEM((N_EXP, CAP, d), jnp.bfloat16),
            pltpu.VMEM((N_DEV, E_LOC, CAP, d), jnp.bfloat16),
            pltpu.VMEM((N_DEV, E_LOC, CAP, h), jnp.bfloat16),
            pltpu.VMEM((N_EXP, CAP, h), jnp.bfloat16),
            pltpu.SemaphoreType.DMA((N_DEV,)),
            pltpu.SemaphoreType.DMA((N_DEV,)),
            pltpu.SemaphoreType.DMA((N_DEV,)),
            pltpu.SemaphoreType.DMA((N_DEV,)),
            pltpu.SemaphoreType.DMA,
            pltpu.SemaphoreType.DMA,
        ],
        compiler_params=pltpu.CompilerParams(
            has_side_effects=True,
            vmem_limit_bytes=52 * 1024 * 1024,
        ),
    )(x_bf, p_bf, k_row, k_col, sW, eW)

    return out_bf.astype(jnp.float32)


# device time: 229206 ns/iter; 1.6010x vs baseline; 1.4061x over previous
import jax
import jax.numpy as jnp
from jax import lax
from jax.experimental import pallas as pl
from jax.experimental.pallas import tpu as pltpu

N_DEV = 32
E_LOC = 4
N_EXP = 128
CAPD = 112
DW = 640
BLK = 8
N_BLK = N_DEV // BLK


def kernel(x, router_W, route_idx, expert_W, shared_W):
    n_tok, d = x.shape
    h = shared_W.shape[1]

    scores = x @ router_W
    probs = jax.nn.softmax(scores, axis=-1)
    e = route_idx[:, 0]
    onehot_e = (e[:, None] == jnp.arange(N_EXP)[None, :])
    p_tok = jnp.sum(probs * onehot_e, axis=1, keepdims=True)

    dest = e // E_LOC
    le = (e % E_LOC).astype(jnp.bfloat16)
    ohd = (dest[:, None] == jnp.arange(N_DEV)[None, :]).astype(jnp.int32)
    pos = jnp.sum((jnp.cumsum(ohd, axis=0) - ohd) * ohd, axis=1)
    k = jnp.where(pos < CAPD, dest * CAPD + pos, -1).astype(jnp.int32)

    k_row = k[None, :]
    k_col = k[:, None]
    x_bf = x.astype(jnp.bfloat16)
    xs = (x * p_tok).astype(jnp.bfloat16)
    xs_aug = jnp.concatenate(
        [xs, jnp.broadcast_to(le[:, None], (n_tok, DW - d))], axis=1)
    sW = shared_W.astype(jnp.bfloat16)
    eW = expert_W.astype(jnp.bfloat16)

    def body(xa_ref, x_ref, krow_ref, kcol_ref, sW_ref, eW_ref, out_ref,
             disp_ref, r_ref, y_ref, z_ref,
             send1, recv1, send2, recv2, cp_sem1, cp_sem2):
        me = lax.axis_index("i")
        xa = xa_ref[...]
        krow = krow_ref[...]

        def build_send(t, carry):
            dd = lax.rem(me + 1 + t, N_DEV)
            oh = (krow == lax.broadcasted_iota(jnp.int32, (CAPD, n_tok), 0)
                  + dd * CAPD).astype(jnp.bfloat16)
            chunk = jnp.dot(oh, xa, preferred_element_type=jnp.float32)
            disp_ref[pl.ds(dd, 1)] = (
                chunk.astype(jnp.bfloat16).reshape(1, CAPD, DW))

            @pl.when(dd != me)
            def _():
                pltpu.make_async_remote_copy(
                    src_ref=disp_ref.at[dd],
                    dst_ref=r_ref.at[me],
                    send_sem=send1.at[dd],
                    recv_sem=recv1.at[me],
                    device_id=dd,
                    device_id_type=pl.DeviceIdType.LOGICAL,
                ).start()

            @pl.when(dd == me)
            def _():
                cp = pltpu.make_async_copy(
                    disp_ref.at[dd], r_ref.at[dd], cp_sem1)
                cp.start()
                cp.wait()
            return carry
        lax.fori_loop(0, N_DEV, build_send, 0)

        sh = jnp.dot(x_ref[...], sW_ref[...],
                     preferred_element_type=jnp.float32)
        out_ref[...] = sh.astype(jnp.bfloat16)

        for b in range(N_BLK):
            def wait_disp(j, carry):
                s = b * BLK + j

                @pl.when(s != me)
                def _():
                    pltpu.make_async_remote_copy(
                        src_ref=disp_ref.at[0],
                        dst_ref=r_ref.at[s],
                        send_sem=send1.at[s],
                        recv_sem=recv1.at[s],
                        device_id=me,
                        device_id_type=pl.DeviceIdType.LOGICAL,
                    ).wait_recv()
                return carry
            lax.fori_loop(0, BLK, wait_disp, 0)

            flat = r_ref[pl.ds(b * BLK, BLK), :, :].reshape(BLK * CAPD, DW)
            xin = flat[:, :d]
            lev = flat[:, d:d + 1]
            yv = jnp.zeros((BLK * CAPD, h), jnp.float32)
            for lei in range(E_LOC):
                mask = (lev == lei).astype(jnp.bfloat16)
                yv = yv + jnp.dot(xin * mask, eW_ref[lei],
                                  preferred_element_type=jnp.float32)
            y_ref[pl.ds(b * BLK, BLK), :, :] = (
                yv.astype(jnp.bfloat16).reshape(BLK, CAPD, h))

            def send_ret(j, carry):
                s = b * BLK + j

                @pl.when(s != me)
                def _():
                    pltpu.make_async_remote_copy(
                        src_ref=y_ref.at[s],
                        dst_ref=z_ref.at[me],
                        send_sem=send2.at[s],
                        recv_sem=recv2.at[me],
                        device_id=s,
                        device_id_type=pl.DeviceIdType.LOGICAL,
                    ).start()

                @pl.when(s == me)
                def _():
                    cp = pltpu.make_async_copy(
                        y_ref.at[s], z_ref.at[s], cp_sem2)
                    cp.start()
                    cp.wait()
                return carry
            lax.fori_loop(0, BLK, send_ret, 0)

        kcol = kcol_ref[...]
        for b in range(N_BLK):
            def wait_ret(j, carry):
                s = b * BLK + j

                @pl.when(s != me)
                def _():
                    pltpu.make_async_remote_copy(
                        src_ref=y_ref.at[0],
                        dst_ref=z_ref.at[s],
                        send_sem=send2.at[s],
                        recv_sem=recv2.at[s],
                        device_id=me,
                        device_id_type=pl.DeviceIdType.LOGICAL,
                    ).wait_recv()
                return carry
            lax.fori_loop(0, BLK, wait_ret, 0)

            def combine(c, carry):
                cc = b * BLK + c
                oh = (kcol == lax.broadcasted_iota(
                    jnp.int32, (n_tok, CAPD), 1)
                    + cc * CAPD).astype(jnp.bfloat16)
                zc = z_ref[cc]
                out_ref[...] = out_ref[...] + jnp.dot(
                    oh, zc,
                    preferred_element_type=jnp.float32).astype(jnp.bfloat16)
                return carry
            lax.fori_loop(0, BLK, combine, 0)

        def wait_sends(t, carry):
            dd = lax.rem(me + 1 + t, N_DEV)
            pltpu.make_async_remote_copy(
                src_ref=disp_ref.at[0],
                dst_ref=r_ref.at[0],
                send_sem=send1.at[dd],
                recv_sem=recv1.at[0],
                device_id=me,
                device_id_type=pl.DeviceIdType.LOGICAL,
            ).wait_send()
            pltpu.make_async_remote_copy(
                src_ref=y_ref.at[0],
                dst_ref=z_ref.at[0],
                send_sem=send2.at[dd],
                recv_sem=recv2.at[0],
                device_id=me,
                device_id_type=pl.DeviceIdType.LOGICAL,
            ).wait_send()
            return carry
        lax.fori_loop(0, N_DEV - 1, wait_sends, 0)

    out_bf = pl.pallas_call(
        body,
        out_shape=jax.ShapeDtypeStruct((n_tok, h), jnp.bfloat16),
        in_specs=[pl.BlockSpec(memory_space=pltpu.VMEM)] * 6,
        out_specs=pl.BlockSpec(memory_space=pltpu.VMEM),
        scratch_shapes=[
            pltpu.VMEM((N_DEV, CAPD, DW), jnp.bfloat16),
            pltpu.VMEM((N_DEV, CAPD, DW), jnp.bfloat16),
            pltpu.VMEM((N_DEV, CAPD, h), jnp.bfloat16),
            pltpu.VMEM((N_DEV, CAPD, h), jnp.bfloat16),
            pltpu.SemaphoreType.DMA((N_DEV,)),
            pltpu.SemaphoreType.DMA((N_DEV,)),
            pltpu.SemaphoreType.DMA((N_DEV,)),
            pltpu.SemaphoreType.DMA((N_DEV,)),
            pltpu.SemaphoreType.DMA,
            pltpu.SemaphoreType.DMA,
        ],
        compiler_params=pltpu.CompilerParams(
            has_side_effects=True,
            vmem_limit_bytes=52 * 1024 * 1024,
        ),
    )(xs_aug, x_bf, k_row, k_col, sW, eW)

    return out_bf.astype(jnp.float32)
